# Initial kernel scaffold; baseline (speedup 1.0000x reference)
import jax
import jax.numpy as jnp
from jax.experimental import pallas as pl


def kernel(x, targets, sub):
    raise NotImplementedError("write your pallas kernel here")


# speedup vs baseline: 1.7479x; 1.7479x over previous
import functools

import jax
import jax.numpy as jnp
from jax.experimental import pallas as pl
from jax.experimental.pallas import tpu as pltpu

_M1, _M2, _A1, _A2, _T = 2.0, 2.0, 2.4, 2.2, 1.0
_N = 4096
_D = 128
_BM = 256


def _loss_kernel(x_ref, t_row_ref, s_row_ref, t_col_ref, s_col_ref,
                 dist_ref, loss_ref, xn_ref, xx_ref, acc_ref):
    i = pl.program_id(0)
    nblocks = pl.num_programs(0)

    @pl.when(i == 0)
    def _init():
        xr = x_ref[...]
        sq = jnp.sum(xr * xr, axis=1, keepdims=True)
        inv = 1.0 / jnp.maximum(jnp.sqrt(sq), 1e-12)
        xn = xr * inv
        xn_ref[...] = xn
        ones = jnp.ones((1, _D), dtype=jnp.float32)
        xx_ref[...] = jax.lax.dot_general(
            ones, xn * xn, (((1,), (1,)), ((), ())),
            preferred_element_type=jnp.float32)
        acc_ref[0, 0] = 0.0

    xn = xn_ref[...]
    r0 = i * _BM
    xn_rows = xn_ref[pl.ds(r0, _BM), :]

    g = jax.lax.dot_general(
        xn_rows, xn, (((1,), (1,)), ((), ())),
        preferred_element_type=jnp.float32)

    xx_cols = xx_ref[...]
    xx_rows = jnp.sum(xn_rows * xn_rows, axis=1, keepdims=True)
    d2 = xx_rows + xx_cols - 2.0 * g
    dist = jnp.sqrt(jnp.maximum(d2, 1e-12))
    dist_ref[...] = dist

    t_row = t_row_ref[...]
    s_row = s_row_ref[...]
    t_col = t_col_ref[...]
    s_col = s_col_ref[...]

    same_t = t_col == t_row
    intra = s_col == s_row
    row_ids = jax.lax.broadcasted_iota(jnp.int32, (_BM, _N), 0) + r0
    col_ids = jax.lax.broadcasted_iota(jnp.int32, (_BM, _N), 1)
    eye = row_ids == col_ids

    intra_f = intra.astype(jnp.float32)
    is_pos_f = (same_t & (~eye)).astype(jnp.float32)
    is_neg_f = (~same_t).astype(jnp.float32)

    mma = jnp.where(intra, _M1 - _A1, _M2 - _A2)
    apv = jnp.maximum(dist + mma, 0.0) * is_pos_f
    apv_i = jnp.sum(apv * intra_f, axis=1)
    apv_all = jnp.sum(apv, axis=1)
    cnt_i = jnp.sum(is_pos_f * intra_f, axis=1)
    cnt_all = jnp.sum(is_pos_f, axis=1)
    loss_ap = (apv_i / (cnt_i + 1e-5)
               + (apv_all - apv_i) / ((cnt_all - cnt_i) + 1e-5))

    di = _A1 - dist
    dd = jnp.where(intra, di, di - (_A1 - _A2))
    less_f = is_neg_f * (dd > 0.0).astype(jnp.float32)
    e = jnp.exp(_T * di)
    e = jnp.where(intra, e, e * jnp.exp(jnp.float32(-_T * (_A1 - _A2))))
    w = e * less_f
    num = dd * w
    w_i = jnp.sum(w * intra_f, axis=1)
    w_all = jnp.sum(w, axis=1)
    num_i = jnp.sum(num * intra_f, axis=1)
    num_all = jnp.sum(num, axis=1)
    loss_an = (num_i / (w_i + 1e-5)
               + (num_all - num_i) / ((w_all - w_i) + 1e-5))

    acc_ref[0, 0] += jnp.sum(loss_ap + loss_an)

    @pl.when(i == nblocks - 1)
    def _final():
        loss_ref[...] = jnp.full((1, 1), acc_ref[0, 0] / jnp.float32(_N),
                                 dtype=jnp.float32)


@jax.jit
def kernel(x, targets, sub):
    t_row = targets.reshape(1, _N).astype(jnp.int32)
    s_row = sub.reshape(1, _N).astype(jnp.int32)
    t_col = targets.reshape(_N, 1).astype(jnp.int32)
    s_col = sub.reshape(_N, 1).astype(jnp.int32)

    grid = (_N // _BM,)
    dist, loss = pl.pallas_call(
        _loss_kernel,
        grid=grid,
        in_specs=[
            pl.BlockSpec((_N, _D), lambda i: (0, 0)),
            pl.BlockSpec((1, _N), lambda i: (0, 0)),
            pl.BlockSpec((1, _N), lambda i: (0, 0)),
            pl.BlockSpec((_BM, 1), lambda i: (i, 0)),
            pl.BlockSpec((_BM, 1), lambda i: (i, 0)),
        ],
        out_specs=[
            pl.BlockSpec((_BM, _N), lambda i: (i, 0)),
            pl.BlockSpec((1, 1), lambda i: (0, 0)),
        ],
        out_shape=[
            jax.ShapeDtypeStruct((_N, _N), jnp.float32),
            jax.ShapeDtypeStruct((1, 1), jnp.float32),
        ],
        scratch_shapes=[
            pltpu.VMEM((_N, _D), jnp.float32),
            pltpu.VMEM((1, _N), jnp.float32),
            pltpu.SMEM((1, 1), jnp.float32),
        ],
    )(x, t_row, s_row, t_col, s_col)
    return loss.reshape(()), dist


# speedup vs baseline: 2.5240x; 1.4440x over previous
import jax
import jax.numpy as jnp
from jax.experimental import pallas as pl
from jax.experimental.pallas import tpu as pltpu

_M1, _M2, _A1, _A2, _T = 2.0, 2.0, 2.4, 2.2, 1.0
_N = 4096
_D = 128
_BM = 256


def _dot(a, b, dims):
    return jax.lax.dot_general(a, b, (dims, ((), ())),
                               preferred_element_type=jnp.float32)


def _loss_kernel(x_ref, t_row_ref, s_row_ref, t_col_ref, s_col_ref,
                 dist_ref, loss_ref, xn_ref, xx_ref, cnt_ref, s2_ref, acc_ref):
    i = pl.program_id(0)
    nblocks = pl.num_programs(0)

    @pl.when(i == 0)
    def _init():
        xr = x_ref[...]
        sq = jnp.sum(xr * xr, axis=1, keepdims=True)
        inv = 1.0 / jnp.maximum(jnp.sqrt(sq), 1e-12)
        xn = xr * inv
        xn_ref[...] = xn
        ones_d = jnp.ones((1, _D), dtype=jnp.float32)
        xx_ref[...] = _dot(ones_d, xn * xn, ((1,), (1,)))

        tc = t_col_ref[...]
        sc = s_col_ref[...]
        th = (tc == jax.lax.broadcasted_iota(jnp.int32, (1, 128), 1)
              ).astype(jnp.float32)
        kh = ((tc * 2 + sc) == jax.lax.broadcasted_iota(jnp.int32, (1, 256), 1)
              ).astype(jnp.float32)
        ones_n = jnp.ones((1, _N), dtype=jnp.float32)
        ccls = _dot(ones_n, th, ((1,), (0,)))
        cpair = _dot(ones_n, kh, ((1,), (0,)))
        classcnt = _dot(th, ccls, ((1,), (1,)))
        paircnt = _dot(kh, cpair, ((1,), (1,)))
        cnt_ref[:, 0:1] = paircnt - 1.0
        cnt_ref[:, 1:2] = classcnt - paircnt
        s2_ref[...] = (sc == jax.lax.broadcasted_iota(jnp.int32, (1, 2), 1)
                       ).astype(jnp.float32)
        acc_ref[0, 0] = 0.0

    r0 = i * _BM
    xn = xn_ref[...]
    xn_rows = xn_ref[pl.ds(r0, _BM), :]
    g = _dot(xn_rows, xn, ((1,), (1,)))

    xx_rows = jnp.sum(xn_rows * xn_rows, axis=1, keepdims=True)
    d2 = (xx_ref[...] - 2.0 * g) + xx_rows
    dist = jnp.sqrt(jnp.maximum(d2, 1e-12))
    dist_ref[...] = dist

    t_row = t_row_ref[...]
    s_row = s_row_ref[...]
    tcb = t_col_ref[pl.ds(r0, _BM), :]
    scb = s_col_ref[pl.ds(r0, _BM), :]

    neq = tcb != t_row
    intra = scb == s_row
    alpha = jnp.where(intra, _A1, _A2)

    apv = jnp.maximum(jnp.where(neq, -1.0, (dist + _M1) - alpha), 0.0)

    dd = alpha - dist
    e = jnp.exp(dd)
    w = jnp.where((dd > 0.0) & neq, e, 0.0)
    num = dd * w

    s2 = s2_ref[...]
    a2 = _dot(apv, s2, ((1,), (0,)))
    w2 = _dot(w, s2, ((1,), (0,)))
    n2 = _dot(num, s2, ((1,), (0,)))

    sint = scb == 0
    ap_i = jnp.where(sint, a2[:, 0:1], a2[:, 1:2])
    ap_c = (a2[:, 0:1] + a2[:, 1:2]) - ap_i
    w_i = jnp.where(sint, w2[:, 0:1], w2[:, 1:2])
    w_c = (w2[:, 0:1] + w2[:, 1:2]) - w_i
    n_i = jnp.where(sint, n2[:, 0:1], n2[:, 1:2])
    n_c = (n2[:, 0:1] + n2[:, 1:2]) - n_i

    cnt = cnt_ref[pl.ds(r0, _BM), :]
    row_loss = (ap_i / (cnt[:, 0:1] + 1e-5) + ap_c / (cnt[:, 1:2] + 1e-5)
                + n_i / (w_i + 1e-5) + n_c / (w_c + 1e-5))
    acc_ref[0, 0] += jnp.sum(row_loss)

    @pl.when(i == nblocks - 1)
    def _final():
        loss_ref[...] = jnp.full((1, 1), acc_ref[0, 0] / jnp.float32(_N),
                                 dtype=jnp.float32)


@jax.jit
def kernel(x, targets, sub):
    t_row = targets.reshape(1, _N).astype(jnp.int32)
    s_row = sub.reshape(1, _N).astype(jnp.int32)
    t_col = targets.reshape(_N, 1).astype(jnp.int32)
    s_col = sub.reshape(_N, 1).astype(jnp.int32)

    grid = (_N // _BM,)
    dist, loss = pl.pallas_call(
        _loss_kernel,
        grid=grid,
        in_specs=[
            pl.BlockSpec((_N, _D), lambda i: (0, 0)),
            pl.BlockSpec((1, _N), lambda i: (0, 0)),
            pl.BlockSpec((1, _N), lambda i: (0, 0)),
            pl.BlockSpec((_N, 1), lambda i: (0, 0)),
            pl.BlockSpec((_N, 1), lambda i: (0, 0)),
        ],
        out_specs=[
            pl.BlockSpec((_BM, _N), lambda i: (i, 0)),
            pl.BlockSpec((1, 1), lambda i: (0, 0)),
        ],
        out_shape=[
            jax.ShapeDtypeStruct((_N, _N), jnp.float32),
            jax.ShapeDtypeStruct((1, 1), jnp.float32),
        ],
        scratch_shapes=[
            pltpu.VMEM((_N, _D), jnp.float32),
            pltpu.VMEM((1, _N), jnp.float32),
            pltpu.VMEM((_N, 2), jnp.float32),
            pltpu.VMEM((_N, 2), jnp.float32),
            pltpu.SMEM((1, 1), jnp.float32),
        ],
    )(x, t_row, s_row, t_col, s_col)
    return loss.reshape(()), dist


# speedup vs baseline: 2.7288x; 1.0811x over previous
import jax
import jax.numpy as jnp
from jax.experimental import pallas as pl
from jax.experimental.pallas import tpu as pltpu

_M1, _M2, _A1, _A2, _T = 2.0, 2.0, 2.4, 2.2, 1.0
_N = 4096
_D = 128
_BM = 512


def _dot(a, b, dims):
    return jax.lax.dot_general(a, b, (dims, ((), ())),
                               preferred_element_type=jnp.float32)


def _loss_kernel(x_ref, t_row_ref, s_row_ref, t_col_ref, s_col_ref,
                 dist_ref, loss_ref, xn_ref, xx_ref, cnt_ref, s2_ref, acc_ref):
    i = pl.program_id(0)
    nblocks = pl.num_programs(0)

    @pl.when(i == 0)
    def _init():
        xr = x_ref[...]
        sq = jnp.sum(xr * xr, axis=1, keepdims=True)
        inv = 1.0 / jnp.maximum(jnp.sqrt(sq), 1e-12)
        xn = xr * inv
        xn_ref[...] = xn
        ones_d = jnp.ones((1, _D), dtype=jnp.float32)
        xx_ref[...] = _dot(ones_d, xn * xn, ((1,), (1,)))

        tc = t_col_ref[...]
        sc = s_col_ref[...]
        th = (tc == jax.lax.broadcasted_iota(jnp.int32, (1, 128), 1)
              ).astype(jnp.float32)
        kh = ((tc * 2 + sc) == jax.lax.broadcasted_iota(jnp.int32, (1, 256), 1)
              ).astype(jnp.float32)
        ones_n = jnp.ones((1, _N), dtype=jnp.float32)
        ccls = _dot(ones_n, th, ((1,), (0,)))
        cpair = _dot(ones_n, kh, ((1,), (0,)))
        classcnt = _dot(th, ccls, ((1,), (1,)))
        paircnt = _dot(kh, cpair, ((1,), (1,)))
        cnt_ref[:, 0:1] = paircnt - 1.0
        cnt_ref[:, 1:2] = classcnt - paircnt
        s2_ref[...] = (sc == jax.lax.broadcasted_iota(jnp.int32, (1, 2), 1)
                       ).astype(jnp.float32)
        acc_ref[0, 0] = 0.0

    r0 = i * _BM
    xn = xn_ref[...]
    xn_rows = xn_ref[pl.ds(r0, _BM), :]
    g = _dot(xn_rows, xn, ((1,), (1,)))

    xx_rows = jnp.sum(xn_rows * xn_rows, axis=1, keepdims=True)
    d2 = (xx_ref[...] - 2.0 * g) + xx_rows
    dist = jnp.sqrt(jnp.maximum(d2, 1e-12))
    dist_ref[...] = dist

    t_row = t_row_ref[...]
    s_row = s_row_ref[...]
    tcb = t_col_ref[pl.ds(r0, _BM), :]
    scb = s_col_ref[pl.ds(r0, _BM), :]

    neq = tcb != t_row
    intra = scb == s_row
    alpha = jnp.where(intra, _A1, _A2)
    u = alpha - dist

    apv = jnp.maximum(jnp.where(neq, -1.0, _M1 - u), 0.0)

    e = jnp.exp(u)
    w = jnp.where((u > 0.0) & neq, e, 0.0)
    num = u * w

    s2 = s2_ref[...]
    a2 = _dot(apv, s2, ((1,), (0,)))
    w2 = _dot(w, s2, ((1,), (0,)))
    n2 = _dot(num, s2, ((1,), (0,)))

    sint = scb == 0
    ap_i = jnp.where(sint, a2[:, 0:1], a2[:, 1:2])
    ap_c = (a2[:, 0:1] + a2[:, 1:2]) - ap_i
    w_i = jnp.where(sint, w2[:, 0:1], w2[:, 1:2])
    w_c = (w2[:, 0:1] + w2[:, 1:2]) - w_i
    n_i = jnp.where(sint, n2[:, 0:1], n2[:, 1:2])
    n_c = (n2[:, 0:1] + n2[:, 1:2]) - n_i

    cnt = cnt_ref[pl.ds(r0, _BM), :]
    row_loss = (ap_i / (cnt[:, 0:1] + 1e-5) + ap_c / (cnt[:, 1:2] + 1e-5)
                + n_i / (w_i + 1e-5) + n_c / (w_c + 1e-5))
    acc_ref[0, 0] += jnp.sum(row_loss)

    @pl.when(i == nblocks - 1)
    def _final():
        loss_ref[...] = jnp.full((1, 1), acc_ref[0, 0] / jnp.float32(_N),
                                 dtype=jnp.float32)


@jax.jit
def kernel(x, targets, sub):
    t_row = targets.reshape(1, _N).astype(jnp.int32)
    s_row = sub.reshape(1, _N).astype(jnp.int32)
    t_col = targets.reshape(_N, 1).astype(jnp.int32)
    s_col = sub.reshape(_N, 1).astype(jnp.int32)

    grid = (_N // _BM,)
    dist, loss = pl.pallas_call(
        _loss_kernel,
        grid=grid,
        in_specs=[
            pl.BlockSpec((_N, _D), lambda i: (0, 0)),
            pl.BlockSpec((1, _N), lambda i: (0, 0)),
            pl.BlockSpec((1, _N), lambda i: (0, 0)),
            pl.BlockSpec((_N, 1), lambda i: (0, 0)),
            pl.BlockSpec((_N, 1), lambda i: (0, 0)),
        ],
        out_specs=[
            pl.BlockSpec((_BM, _N), lambda i: (i, 0)),
            pl.BlockSpec((1, 1), lambda i: (0, 0)),
        ],
        out_shape=[
            jax.ShapeDtypeStruct((_N, _N), jnp.float32),
            jax.ShapeDtypeStruct((1, 1), jnp.float32),
        ],
        scratch_shapes=[
            pltpu.VMEM((_N, _D), jnp.float32),
            pltpu.VMEM((1, _N), jnp.float32),
            pltpu.VMEM((_N, 2), jnp.float32),
            pltpu.VMEM((_N, 2), jnp.float32),
            pltpu.SMEM((1, 1), jnp.float32),
        ],
    )(x, t_row, s_row, t_col, s_col)
    return loss.reshape(()), dist


# speedup vs baseline: 3.3689x; 1.2346x over previous
import jax
import jax.numpy as jnp
from jax.experimental import pallas as pl
from jax.experimental.pallas import tpu as pltpu

_M1, _M2, _A1, _A2, _T = 2.0, 2.0, 2.4, 2.2, 1.0
_N = 4096
_D = 128
_K = _D + 2
_BM = 512
_LOG2E = 1.4426950408889634


def _dot(a, b, dims):
    return jax.lax.dot_general(a, b, (dims, ((), ())),
                               preferred_element_type=jnp.float32)


def _loss_kernel(x_ref, t_row_ref, s_row_ref, t_col_ref, s_col_ref,
                 dist_ref, loss_ref, lhs_ref, rhs_ref, cnt_ref, s2_ref,
                 acc_ref):
    i = pl.program_id(0)
    nblocks = pl.num_programs(0)

    @pl.when(i == 0)
    def _init():
        xr = x_ref[...]
        sq = jnp.sum(xr * xr, axis=1, keepdims=True)
        inv = 1.0 / jnp.maximum(jnp.sqrt(sq), 1e-12)
        xn = xr * inv
        xq = jnp.sum(xn * xn, axis=1, keepdims=True)
        one_col = jnp.ones((_N, 1), dtype=jnp.float32)
        lhs_ref[:, 0:_D] = xn
        lhs_ref[:, _D:_D + 1] = one_col
        lhs_ref[:, _D + 1:_K] = xq
        rhs_ref[:, 0:_D] = xn * (-2.0)
        rhs_ref[:, _D:_D + 1] = xq
        rhs_ref[:, _D + 1:_K] = one_col

        tc = t_col_ref[...]
        sc = s_col_ref[...]
        th = (tc == jax.lax.broadcasted_iota(jnp.int32, (1, 128), 1)
              ).astype(jnp.float32)
        kh = ((tc * 2 + sc) == jax.lax.broadcasted_iota(jnp.int32, (1, 256), 1)
              ).astype(jnp.float32)
        ones_n = jnp.ones((1, _N), dtype=jnp.float32)
        ccls = _dot(ones_n, th, ((1,), (0,)))
        cpair = _dot(ones_n, kh, ((1,), (0,)))
        classcnt = _dot(th, ccls, ((1,), (1,)))
        paircnt = _dot(kh, cpair, ((1,), (1,)))
        cnt_ref[:, 0:1] = paircnt - 1.0
        cnt_ref[:, 1:2] = classcnt - paircnt
        s2_ref[...] = (sc == jax.lax.broadcasted_iota(jnp.int32, (1, 2), 1)
                       ).astype(jnp.float32)
        acc_ref[0, 0] = 0.0

    r0 = i * _BM
    lhs_blk = lhs_ref[pl.ds(r0, _BM), :]
    d2 = _dot(lhs_blk, rhs_ref[...], ((1,), (1,)))
    m = jnp.maximum(d2, 1e-12)
    dist = m * jax.lax.rsqrt(m)
    dist_ref[...] = dist

    t_row = t_row_ref[...]
    s_row = s_row_ref[...]
    tcb = t_col_ref[pl.ds(r0, _BM), :]
    scb = s_col_ref[pl.ds(r0, _BM), :]

    neq = tcb != t_row
    intra = scb == s_row
    alpha = jnp.where(intra, _A1, _A2)
    u = alpha - dist

    apv = jnp.maximum(jnp.where(neq, -1.0, _M1 - u), 0.0)

    e = jax.lax.exp2(u * _LOG2E)
    w = jnp.where((u > 0.0) & neq, e, 0.0)
    num = u * w

    s2 = s2_ref[...]
    a2 = _dot(apv, s2, ((1,), (0,)))
    w2 = _dot(w, s2, ((1,), (0,)))
    n2 = _dot(num, s2, ((1,), (0,)))

    sint = scb == 0
    ap_i = jnp.where(sint, a2[:, 0:1], a2[:, 1:2])
    ap_c = (a2[:, 0:1] + a2[:, 1:2]) - ap_i
    w_i = jnp.where(sint, w2[:, 0:1], w2[:, 1:2])
    w_c = (w2[:, 0:1] + w2[:, 1:2]) - w_i
    n_i = jnp.where(sint, n2[:, 0:1], n2[:, 1:2])
    n_c = (n2[:, 0:1] + n2[:, 1:2]) - n_i

    cnt = cnt_ref[pl.ds(r0, _BM), :]
    row_loss = (ap_i / (cnt[:, 0:1] + 1e-5) + ap_c / (cnt[:, 1:2] + 1e-5)
                + n_i / (w_i + 1e-5) + n_c / (w_c + 1e-5))
    acc_ref[0, 0] += jnp.sum(row_loss)

    @pl.when(i == nblocks - 1)
    def _final():
        loss_ref[...] = jnp.full((1, 1), acc_ref[0, 0] / jnp.float32(_N),
                                 dtype=jnp.float32)


@jax.jit
def kernel(x, targets, sub):
    t_row = targets.reshape(1, _N).astype(jnp.int32)
    s_row = sub.reshape(1, _N).astype(jnp.int32)
    t_col = targets.reshape(_N, 1).astype(jnp.int32)
    s_col = sub.reshape(_N, 1).astype(jnp.int32)

    grid = (_N // _BM,)
    dist, loss = pl.pallas_call(
        _loss_kernel,
        grid=grid,
        in_specs=[
            pl.BlockSpec((_N, _D), lambda i: (0, 0)),
            pl.BlockSpec((1, _N), lambda i: (0, 0)),
            pl.BlockSpec((1, _N), lambda i: (0, 0)),
            pl.BlockSpec((_N, 1), lambda i: (0, 0)),
            pl.BlockSpec((_N, 1), lambda i: (0, 0)),
        ],
        out_specs=[
            pl.BlockSpec((_BM, _N), lambda i: (i, 0)),
            pl.BlockSpec((1, 1), lambda i: (0, 0)),
        ],
        out_shape=[
            jax.ShapeDtypeStruct((_N, _N), jnp.float32),
            jax.ShapeDtypeStruct((1, 1), jnp.float32),
        ],
        scratch_shapes=[
            pltpu.VMEM((_N, _K), jnp.float32),
            pltpu.VMEM((_N, _K), jnp.float32),
            pltpu.VMEM((_N, 2), jnp.float32),
            pltpu.VMEM((_N, 2), jnp.float32),
            pltpu.SMEM((1, 1), jnp.float32),
        ],
    )(x, t_row, s_row, t_col, s_col)
    return loss.reshape(()), dist


# speedup vs baseline: 3.3810x; 1.0036x over previous
import jax
import jax.numpy as jnp
from jax.experimental import pallas as pl
from jax.experimental.pallas import tpu as pltpu

_M1, _M2, _A1, _A2, _T = 2.0, 2.0, 2.4, 2.2, 1.0
_N = 4096
_D = 128
_K = _D + 2
_BM = 512
_LOG2E = 1.4426950408889634


def _dot(a, b, dims):
    return jax.lax.dot_general(a, b, (dims, ((), ())),
                               preferred_element_type=jnp.float32)


def _loss_kernel(x_ref, t_row_ref, s_row_ref, t_col_ref, s_col_ref,
                 dist_ref, loss_ref, lhs_ref, rhs_ref, cnt_ref, s2_ref,
                 acc_ref):
    i = pl.program_id(0)
    nblocks = pl.num_programs(0)

    @pl.when(i == 0)
    def _init():
        xr = x_ref[...]
        sq = jnp.sum(xr * xr, axis=1, keepdims=True)
        inv = 1.0 / jnp.maximum(jnp.sqrt(sq), 1e-12)
        xn = xr * inv
        xq = jnp.sum(xn * xn, axis=1, keepdims=True)
        one_col = jnp.ones((_N, 1), dtype=jnp.float32)
        lhs_ref[:, 0:_D] = xn
        lhs_ref[:, _D:_D + 1] = one_col
        lhs_ref[:, _D + 1:_K] = xq
        rhs_ref[:, 0:_D] = xn * (-2.0)
        rhs_ref[:, _D:_D + 1] = xq
        rhs_ref[:, _D + 1:_K] = one_col

        tc = t_col_ref[...]
        sc = s_col_ref[...]
        th = (tc == jax.lax.broadcasted_iota(jnp.int32, (1, 128), 1)
              ).astype(jnp.float32)
        kh = ((tc * 2 + sc) == jax.lax.broadcasted_iota(jnp.int32, (1, 256), 1)
              ).astype(jnp.float32)
        ones_n = jnp.ones((1, _N), dtype=jnp.float32)
        ccls = _dot(ones_n, th, ((1,), (0,)))
        cpair = _dot(ones_n, kh, ((1,), (0,)))
        classcnt = _dot(th, ccls, ((1,), (1,)))
        paircnt = _dot(kh, cpair, ((1,), (1,)))
        cnt_ref[:, 0:1] = paircnt - 1.0
        cnt_ref[:, 1:2] = classcnt - paircnt
        s2_ref[...] = (sc == jax.lax.broadcasted_iota(jnp.int32, (1, 2), 1)
                       ).astype(jnp.float32)
        acc_ref[0, 0] = 0.0

    r0 = i * _BM
    lhs_blk = lhs_ref[pl.ds(r0, _BM), :]
    d2 = _dot(lhs_blk, rhs_ref[...], ((1,), (1,)))
    m = jnp.maximum(d2, 1e-12)
    dist = m * jax.lax.rsqrt(m)
    dist_ref[...] = dist

    t_row = t_row_ref[...]
    s_row = s_row_ref[...]
    tcb = t_col_ref[pl.ds(r0, _BM), :]
    scb = s_col_ref[pl.ds(r0, _BM), :]

    neq = tcb != t_row
    _DA = _A1 - _A2
    sf_row = s_row.astype(jnp.float32)
    sf_col = scb.astype(jnp.float32)
    c1 = _A1 - _DA * sf_row
    c2 = (2.0 * _DA) * sf_row - _DA
    alpha = c1 + sf_col * c2
    u = alpha - dist

    apv = jnp.maximum(jnp.where(neq, -1.0, _M1 - u), 0.0)

    e = jax.lax.exp2(u * _LOG2E)
    w = jnp.where((u > 0.0) & neq, e, 0.0)
    num = u * w

    s2 = s2_ref[...]
    a2 = _dot(apv, s2, ((1,), (0,)))
    w2 = _dot(w, s2, ((1,), (0,)))
    n2 = _dot(num, s2, ((1,), (0,)))

    sint = scb == 0
    ap_i = jnp.where(sint, a2[:, 0:1], a2[:, 1:2])
    ap_c = (a2[:, 0:1] + a2[:, 1:2]) - ap_i
    w_i = jnp.where(sint, w2[:, 0:1], w2[:, 1:2])
    w_c = (w2[:, 0:1] + w2[:, 1:2]) - w_i
    n_i = jnp.where(sint, n2[:, 0:1], n2[:, 1:2])
    n_c = (n2[:, 0:1] + n2[:, 1:2]) - n_i

    cnt = cnt_ref[pl.ds(r0, _BM), :]
    row_loss = (ap_i / (cnt[:, 0:1] + 1e-5) + ap_c / (cnt[:, 1:2] + 1e-5)
                + n_i / (w_i + 1e-5) + n_c / (w_c + 1e-5))
    acc_ref[0, 0] += jnp.sum(row_loss)

    @pl.when(i == nblocks - 1)
    def _final():
        loss_ref[...] = jnp.full((1, 1), acc_ref[0, 0] / jnp.float32(_N),
                                 dtype=jnp.float32)


@jax.jit
def kernel(x, targets, sub):
    t_row = targets.reshape(1, _N).astype(jnp.int32)
    s_row = sub.reshape(1, _N).astype(jnp.int32)
    t_col = targets.reshape(_N, 1).astype(jnp.int32)
    s_col = sub.reshape(_N, 1).astype(jnp.int32)

    grid = (_N // _BM,)
    dist, loss = pl.pallas_call(
        _loss_kernel,
        grid=grid,
        in_specs=[
            pl.BlockSpec((_N, _D), lambda i: (0, 0)),
            pl.BlockSpec((1, _N), lambda i: (0, 0)),
            pl.BlockSpec((1, _N), lambda i: (0, 0)),
            pl.BlockSpec((_N, 1), lambda i: (0, 0)),
            pl.BlockSpec((_N, 1), lambda i: (0, 0)),
        ],
        out_specs=[
            pl.BlockSpec((_BM, _N), lambda i: (i, 0)),
            pl.BlockSpec((1, 1), lambda i: (0, 0)),
        ],
        out_shape=[
            jax.ShapeDtypeStruct((_N, _N), jnp.float32),
            jax.ShapeDtypeStruct((1, 1), jnp.float32),
        ],
        scratch_shapes=[
            pltpu.VMEM((_N, _K), jnp.float32),
            pltpu.VMEM((_N, _K), jnp.float32),
            pltpu.VMEM((_N, 2), jnp.float32),
            pltpu.VMEM((_N, 2), jnp.float32),
            pltpu.SMEM((1, 1), jnp.float32),
        ],
    )(x, t_row, s_row, t_col, s_col)
    return loss.reshape(()), dist
